# Initial kernel scaffold; baseline (speedup 1.0000x reference)
import jax
import jax.numpy as jnp
from jax.experimental import pallas as pl


def kernel(x, params, edge_index):
    raise NotImplementedError("write your pallas kernel here")


# speedup vs baseline: 2.1790x; 2.1790x over previous
import functools

import jax
import jax.numpy as jnp
from jax import lax
from jax.experimental import pallas as pl
from jax.experimental.pallas import tpu as pltpu
from jax.experimental.pallas import tpu_sc as plsc

N = 10000
E = 320000
NT = 32
NPT = 320
NPAD = NT * NPT
EPT = E // NT
CHUNK = 128
CAPB = ((EPT + CHUNK - 1) // CHUNK) * CHUNK

_mesh = plsc.VectorSubcoreMesh(core_axis_name="c", subcore_axis_name="s")


def _wid():
    return lax.axis_index("s") * 2 + lax.axis_index("c")


BUFW = CHUNK + 16


def _p1_body(src_hbm, dst_hbm, bsrc_hbm, bdst_hbm, cnt_hbm,
             src_v, dst_v, buf_s, buf_d, cstage_v, cnt_s):
    wid = _wid()
    pltpu.sync_copy(src_hbm.at[pl.ds(wid * EPT, EPT)], src_v)
    pltpu.sync_copy(dst_hbm.at[pl.ds(wid * EPT, EPT)], dst_v)

    zi = jnp.zeros((16,), jnp.int32)
    i16 = lax.iota(jnp.int32, 16)

    @pl.loop(0, NT * (BUFW // 16))
    def _(r):
        buf_s[pl.ds(r * 16, 16)] = zi
        buf_d[pl.ds(r * 16, 16)] = zi

    for o in range(NT):
        cnt_s[o] = 0

    @pl.loop(0, EPT // 16)
    def _(g):
        sv = src_v[pl.ds(g * 16, 16)]
        dv = dst_v[pl.ds(g * 16, 16)]
        ov = (dv * 6554) >> 21
        dlv = dv - ov * NPT
        for o in range(NT):
            msk = ov == o
            npc = jnp.sum(msk.astype(jnp.int32))
            c = cnt_s[o]
            pos = c & (CHUNK - 1)
            plsc.store_compressed(buf_s.at[pl.ds(o * BUFW + pos, 16)], sv, mask=msk)
            plsc.store_compressed(buf_d.at[pl.ds(o * BUFW + pos, 16)], dlv, mask=msk)
            cnt_s[o] = c + npc

            @pl.when(pos + npc >= CHUNK)
            def _():
                base = (c >> 7) * CHUNK
                off = (wid * NT + o) * CAPB + base
                pltpu.sync_copy(buf_s.at[pl.ds(o * BUFW, CHUNK)],
                                bsrc_hbm.at[pl.ds(off, CHUNK)])
                pltpu.sync_copy(buf_d.at[pl.ds(o * BUFW, CHUNK)],
                                bdst_hbm.at[pl.ds(off, CHUNK)])
                ts = buf_s[pl.ds(o * BUFW + CHUNK, 16)]
                td = buf_d[pl.ds(o * BUFW + CHUNK, 16)]
                buf_s[pl.ds(o * BUFW, 16)] = ts
                buf_d[pl.ds(o * BUFW, 16)] = td

    for o in range(NT):
        c = cnt_s[o]

        @pl.when((c & (CHUNK - 1)) != 0)
        def _():
            base = (c >> 7) * CHUNK
            off = (wid * NT + o) * CAPB + base
            pltpu.sync_copy(buf_s.at[pl.ds(o * BUFW, CHUNK)],
                            bsrc_hbm.at[pl.ds(off, CHUNK)])
            pltpu.sync_copy(buf_d.at[pl.ds(o * BUFW, CHUNK)],
                            bdst_hbm.at[pl.ds(off, CHUNK)])

        cb = jnp.full((16,), c, jnp.int32)
        plsc.store_compressed(cstage_v.at[pl.ds(o, 16)], cb, mask=i16 == 0)

    pltpu.sync_copy(cstage_v.at[pl.ds(0, NT)], cnt_hbm.at[pl.ds(wid * NT, NT)])


_p1 = pl.kernel(
    _p1_body,
    out_type=(
        jax.ShapeDtypeStruct((NT * NT * CAPB,), jnp.int32),
        jax.ShapeDtypeStruct((NT * NT * CAPB,), jnp.int32),
        jax.ShapeDtypeStruct((NT * NT,), jnp.int32),
    ),
    mesh=_mesh,
    compiler_params=pltpu.CompilerParams(needs_layout_passes=False),
    scratch_types=[
        pltpu.VMEM((EPT,), jnp.int32),
        pltpu.VMEM((EPT,), jnp.int32),
        pltpu.VMEM((NT * BUFW,), jnp.int32),
        pltpu.VMEM((NT * BUFW,), jnp.int32),
        pltpu.VMEM((NT + 16,), jnp.int32),
        pltpu.SMEM((NT,), jnp.int32),
    ],
)


CT = 128


def _seg_count(cnt_v, s, wid, i16):
    row_a = cnt_v[pl.ds(s * NT, 16)]
    row_b = cnt_v[pl.ds(s * NT + 16, 16)]
    half = jnp.where(wid >= 16, row_b, row_a)
    return jnp.sum(jnp.where(i16 == (wid & 15), half, 0))


def _agg_body(CR, table_hbm, bsrc_hbm, bdst_hbm, cnt_hbm, out_hbm,
              acc, srcc, dstc, rows, cnt_v, gsem):
    wid = _wid()
    pltpu.sync_copy(cnt_hbm, cnt_v)

    inf16 = jnp.full((16,), jnp.inf, jnp.float32)
    i16 = lax.iota(jnp.int32, 16)

    @pl.loop(0, (NPT + 1) * (CR // 16))
    def _(i):
        acc[pl.ds(i * 16, 16)] = inf16

    @pl.loop(0, NT)
    def _(s):
        n = _seg_count(cnt_v, s, wid, i16)

        @pl.loop(0, (n + CHUNK - 1) >> 7)
        def _(ci):
            base = ci * CHUNK
            off = (s * NT + wid) * CAPB + base
            pltpu.sync_copy(bsrc_hbm.at[pl.ds(off, CHUNK)], srcc)
            pltpu.sync_copy(bdst_hbm.at[pl.ds(off, CHUNK)], dstc)
            pltpu.async_copy(table_hbm.at[srcc], rows, gsem).wait()
            rem = n - base

            for g in range(CHUNK // 16):
                dv = dstc[pl.ds(g * 16, 16)]
                for lane in range(16):
                    e = g * 16 + lane
                    d = jnp.where(e < rem, dv[lane], NPT)
                    dbase = d * CR
                    for cc in range(CR // 16):
                        sl = pl.ds(dbase + cc * 16, 16)
                        acc[sl] = jnp.minimum(acc[sl],
                                              rows[e, pl.ds(cc * 16, 16)])

    pltpu.sync_copy(acc.at[pl.ds(0, NPT * CR)],
                    out_hbm.at[pl.ds(wid * NPT * CR, NPT * CR)])


@functools.cache
def _make_agg(CR):
    return pl.kernel(
        functools.partial(_agg_body, CR),
        out_type=jax.ShapeDtypeStruct((NPAD * CR,), jnp.float32),
        mesh=_mesh,
        compiler_params=pltpu.CompilerParams(needs_layout_passes=False),
        scratch_types=[
            pltpu.VMEM(((NPT + 1) * CR,), jnp.float32),
            pltpu.VMEM((CHUNK,), jnp.int32),
            pltpu.VMEM((CHUNK,), jnp.int32),
            pltpu.VMEM((CHUNK, CT), jnp.float32),
            pltpu.VMEM((NT * NT,), jnp.int32),
            pltpu.SemaphoreType.DMA,
        ],
    )


def _aggl_body(table_hbm, bsrc_hbm, bdst_hbm, cnt_hbm, out_hbm,
               tbl, acc, srcc, dstc, cnt_v):
    wid = _wid()
    pltpu.sync_copy(cnt_hbm, cnt_v)
    pltpu.sync_copy(table_hbm, tbl)

    inf16 = jnp.full((16,), jnp.inf, jnp.float32)
    i16 = lax.iota(jnp.int32, 16)
    lo4 = i16 < 4

    @pl.loop(0, ((NPT + 4) * 4) // 16)
    def _(i):
        acc[pl.ds(i * 16, 16)] = inf16

    @pl.loop(0, NT)
    def _(s):
        n = _seg_count(cnt_v, s, wid, i16)

        @pl.loop(0, (n + CHUNK - 1) >> 7)
        def _(ci):
            base = ci * CHUNK
            off = (s * NT + wid) * CAPB + base
            pltpu.sync_copy(bsrc_hbm.at[pl.ds(off, CHUNK)], srcc)
            pltpu.sync_copy(bdst_hbm.at[pl.ds(off, CHUNK)], dstc)
            rem = n - base

            for g in range(CHUNK // 16):
                dv = dstc[pl.ds(g * 16, 16)]
                sv = srcc[pl.ds(g * 16, 16)]
                for lane in range(16):
                    e = g * 16 + lane
                    d = jnp.where(e < rem, dv[lane], NPT)
                    msg = tbl[pl.ds(sv[lane] * 4, 16)]
                    msg = jnp.where(lo4, msg, jnp.inf)
                    sl = pl.ds(d * 4, 16)
                    acc[sl] = jnp.minimum(acc[sl], msg)

    pltpu.sync_copy(acc.at[pl.ds(0, NPT * 4)],
                    out_hbm.at[pl.ds(wid * NPT * 4, NPT * 4)])


_aggl = pl.kernel(
    _aggl_body,
    out_type=jax.ShapeDtypeStruct((NPAD * 4,), jnp.float32),
    mesh=_mesh,
    compiler_params=pltpu.CompilerParams(needs_layout_passes=False),
    scratch_types=[
        pltpu.VMEM(((N + 4) * 4,), jnp.float32),
        pltpu.VMEM(((NPT + 4) * 4,), jnp.float32),
        pltpu.VMEM((CHUNK,), jnp.int32),
        pltpu.VMEM((CHUNK,), jnp.int32),
        pltpu.VMEM((NT * NT,), jnp.int32),
    ],
)


def _mask_agg(a):
    return jnp.where(a[:, 0:1] == jnp.inf, 0.0, a)


def _mm(a, w):
    return lax.dot_general(a, w, (((1,), (1,)), ((), ())),
                           preferred_element_type=jnp.float32)


def _bn_relu(z, g, b):
    m = jnp.mean(z, axis=0, keepdims=True)
    v = jnp.mean((z - m) * (z - m), axis=0, keepdims=True)
    return jnp.maximum(g * (z - m) / jnp.sqrt(v + 1e-5) + b, 0.0)


def _sage_block_body(agg_ref, xr_ref, wl_ref, bl_ref, wr_ref, g_ref, b_ref,
                     o_ref):
    ag = _mask_agg(agg_ref[...][:N])
    z = _mm(ag, wl_ref[...]) + bl_ref[...] + _mm(xr_ref[...], wr_ref[...])
    o_ref[...] = _bn_relu(z, g_ref[...], b_ref[...])


def _sage_lin_body(agg_ref, xr_ref, wl_ref, bl_ref, wr_ref, o_ref):
    ag = _mask_agg(agg_ref[...][:N])
    o_ref[...] = _mm(ag, wl_ref[...]) + bl_ref[...] + _mm(xr_ref[...],
                                                          wr_ref[...])


def _shared_body(aggx_ref, aggr_ref, x_ref, rm_ref, wla_ref, wlb_ref, bl_ref,
                 wra_ref, wrb_ref, g_ref, b_ref, o_ref):
    agx = _mask_agg(aggx_ref[...][:N])
    agr = _mask_agg(aggr_ref[...][:N])
    z = (_mm(agx, wla_ref[...]) + _mm(agr, wlb_ref[...]) + bl_ref[...]
         + _mm(x_ref[...], wra_ref[...]) + _mm(rm_ref[...], wrb_ref[...]))
    o_ref[...] = _bn_relu(z, g_ref[...], b_ref[...])


def _label_body(rm_ref, ls_ref, lab_ref):
    rm = rm_ref[...]
    col = lax.broadcasted_iota(jnp.int32, rm.shape, 1)
    valid = col < 3
    ones_col = jnp.ones((rm.shape[1], 1), jnp.float32)

    def bc(v):
        return lax.dot_general(v, ones_col, (((1,), (1,)), ((), ())),
                               precision=lax.Precision.HIGHEST,
                               preferred_element_type=jnp.float32)

    mx = jnp.max(jnp.where(valid, rm, -jnp.inf), axis=1, keepdims=True)
    mxb = bc(mx)
    se = jnp.sum(jnp.where(valid, jnp.exp(rm - mxb), 0.0), axis=1,
                 keepdims=True)
    ls_ref[...] = rm - mxb - bc(jnp.log(se))

    c0 = jnp.sum(jnp.where(col == 0, rm, 0.0), axis=1, keepdims=True)
    c1 = jnp.sum(jnp.where(col == 1, rm, 0.0), axis=1, keepdims=True)
    c2 = jnp.sum(jnp.where(col == 2, rm, 0.0), axis=1, keepdims=True)
    lab_ref[...] = bc(jnp.where((c2 > c0) & (c2 > c1), 1.0, 0.0))


def _head_body(aggh_ref, h_ref, lab_ref,
               pwl_ref, pbl_ref, pwr_ref, pg_ref, pb_ref, pw3_ref, pb3_ref,
               swl_ref, sbl_ref, swr_ref, sg_ref, sb_ref, sw3_ref, sb3_ref,
               pre_ref, suc_ref):
    agh = _mask_agg(aggh_ref[...][:N])
    h = h_ref[...]
    lab = lab_ref[...]
    pre = _bn_relu(_mm(agh, pwl_ref[...]) + pbl_ref[...]
                   + _mm(h, pwr_ref[...]), pg_ref[...], pb_ref[...])
    pre = _mm(pre, pw3_ref[...]) + pb3_ref[...]
    suc = _bn_relu(_mm(agh, swl_ref[...]) + sbl_ref[...]
                   + _mm(h, swr_ref[...]), sg_ref[...], sb_ref[...])
    suc = _mm(suc, sw3_ref[...]) + sb3_ref[...]
    pre_ref[...] = pre * lab
    suc_ref[...] = suc * lab


def _tc(body, *out_shapes):
    return pl.pallas_call(
        body,
        out_shape=tuple(jax.ShapeDtypeStruct(s, jnp.float32)
                        for s in out_shapes),
    )


def kernel(x, params, edge_index):
    p = params
    src = edge_index[0]
    dst = edge_index[1]
    bsrc, bdst, cnt = _p1(src, dst)

    def agg(table, cr):
        return _make_agg(cr)(table, bsrc, bdst, cnt).reshape(NPAD, cr)

    def r2(v):
        return v.reshape(1, -1)

    def padw(w, r, c):
        out = jnp.zeros((r, c), jnp.float32)
        return lax.dynamic_update_slice(out, w, (0, 0))

    aggx = agg(x, 128)
    (rm1,) = _tc(_sage_block_body, (N, 128))(
        aggx, x, p["rm_l1"]["Wl"], r2(p["rm_l1"]["bl"]), p["rm_l1"]["Wr"],
        r2(p["rm_norm1"]["g"]), r2(p["rm_norm1"]["b"]))

    aggr1 = agg(rm1, 128)
    (rm2p,) = _tc(_sage_block_body, (N, 128))(
        aggr1, rm1, padw(p["rm_l2"]["Wl"], 128, 128),
        padw(r2(p["rm_l2"]["bl"]), 1, 128), padw(p["rm_l2"]["Wr"], 128, 128),
        padw(r2(p["rm_norm2"]["g"]), 1, 128), padw(r2(p["rm_norm2"]["b"]),
                                                   1, 128))

    aggr2 = agg(rm2p, 64)
    wl4 = padw(p["rm_l4"]["Wl"], 128, 64)
    wr4 = padw(p["rm_l4"]["Wr"], 128, 128)
    bl4 = padw(r2(p["rm_l4"]["bl"]), 1, 128)
    (rm128,) = _tc(_sage_lin_body, (N, 128))(aggr2, rm2p, wl4, bl4, wr4)

    rm4f = jnp.concatenate(
        [rm128[:, :4], jnp.zeros((4, 4), jnp.float32)]).reshape(-1)
    aggrm4 = _aggl(rm4f, bsrc, bdst, cnt).reshape(NPAD, 4)
    aggrm = jnp.concatenate(
        [aggrm4, jnp.zeros((NPAD, 124), jnp.float32)], axis=1)
    wl_sh = p["shared_l1"]["Wl"]
    wr_sh = p["shared_l1"]["Wr"]
    wlb = padw(wl_sh[:, 128:], 128, 128)
    wrb = padw(wr_sh[:, 128:], 128, 128)
    (h,) = _tc(_shared_body, (N, 128))(
        aggx, aggrm, x, rm128, wl_sh[:, :128], wlb, r2(p["shared_l1"]["bl"]),
        wr_sh[:, :128], wrb, r2(p["shared_norm1"]["g"]),
        r2(p["shared_norm1"]["b"]))

    aggh = agg(h, 128)
    _lspec = pl.BlockSpec((2000, 128), lambda i: (i, 0))
    ls128, lab = pl.pallas_call(
        _label_body,
        grid=(N // 2000,),
        in_specs=[_lspec],
        out_specs=(_lspec, _lspec),
        out_shape=(jax.ShapeDtypeStruct((N, 128), jnp.float32),
                   jax.ShapeDtypeStruct((N, 128), jnp.float32)),
    )(rm128)
    pre, suc = _tc(_head_body, (N, 128), (N, 128))(
        aggh, h, lab,
        p["premove_l1"]["Wl"], r2(p["premove_l1"]["bl"]),
        p["premove_l1"]["Wr"], r2(p["premove_norm1"]["g"]),
        r2(p["premove_norm1"]["b"]),
        jnp.broadcast_to(p["premove_l3"]["W"], (128, 64)),
        jnp.broadcast_to(r2(p["premove_l3"]["b"]), (1, 128)),
        p["sucmove_l1"]["Wl"], r2(p["sucmove_l1"]["bl"]),
        p["sucmove_l1"]["Wr"], r2(p["sucmove_norm1"]["g"]),
        r2(p["sucmove_norm1"]["b"]),
        jnp.broadcast_to(p["sucmove_l3"]["W"], (128, 64)),
        jnp.broadcast_to(r2(p["sucmove_l3"]["b"]), (1, 128)))

    return ls128[:, :3], pre[:, 0], suc[:, 0]
